# baseline (speedup 1.0000x reference)
import functools

import jax
import jax.numpy as jnp
from jax.experimental import pallas as pl
from jax.experimental.pallas import tpu as pltpu

_C = 128
_NEWTON = 6
_DEPTH = 2


def _l2n(v):
    n = jnp.sqrt(jnp.sum(v * v, axis=-1, keepdims=True))
    return v / jnp.maximum(n, 1e-12)


def _f32dot(a, b):
    return jnp.dot(a, b, preferred_element_type=jnp.float32)


def _dotT(a, b, ca, cb):
    return jax.lax.dot_general(a, b, (((ca,), (cb,)), ((), ())),
                               preferred_element_type=jnp.float32)


def _hdot(a, b):
    return jnp.dot(a, b, preferred_element_type=jnp.float32,
                   precision=jax.lax.Precision.HIGHEST)


def _hdotT(a, b, ca, cb):
    return jax.lax.dot_general(a, b, (((ca,), (cb,)), ((), ())),
                               preferred_element_type=jnp.float32,
                               precision=jax.lax.Precision.HIGHEST)


def _amem_kernel(x_ref, M_ref, WvT_ref, WkT_ref, Q_ref, WoutT_ref,
                 out_ref, Mf_ref, M_scr, V_scr, K_scr, X_scr,
                 *, B, C, R, depth):
    c = pl.program_id(0)
    nc = pl.num_programs(0) - 1

    @pl.when(c == 0)
    def _():
        M_scr[...] = M_ref[...]

    row = jax.lax.broadcasted_iota(jnp.int32, (C, C), 0)
    col = jax.lax.broadcasted_iota(jnp.int32, (C, C), 1)


    prev = (c - 1) & 1
    Vb = [V_scr[prev, b] for b in range(B)]
    Kb = [K_scr[prev, b] for b in range(B)]
    Xb = [X_scr[prev, b] for b in range(B)]
    M0 = [M_scr[b] for b in range(B)]

    cur_par = c & 1
    x_all = jnp.concatenate([x_ref[b] for b in range(B)], axis=0)
    V_all_n = _f32dot(x_all, WvT_ref[...])
    K_all_n = _l2n(_f32dot(x_all, WkT_ref[...]))
    eye = jnp.where(col == row, 1.0, 0.0)
    for b in range(B):
        Kn = K_all_n[b * C:(b + 1) * C]
        Ab = jnp.where(col < row, _hdotT(Kn, Kn, 1, 1), 0.0)
        Xn = eye - Ab
        for it in range(_NEWTON):
            dot = _hdot if it == _NEWTON - 1 else _f32dot
            Xn = 2.0 * Xn - dot(Xn, Xn + dot(Ab, Xn))
        V_scr[cur_par, b] = V_all_n[b * C:(b + 1) * C]
        K_scr[cur_par, b] = Kn
        X_scr[cur_par, b] = Xn

    Ub = [_hdot(Xb[b], Vb[b] - _hdotT(Kb[b], M0[b], 1, 1))
          for b in range(B)]
    live = c > 0
    for b in range(B):
        M1 = M0[b] + _hdotT(Ub[b], Kb[b], 0, 0)
        M1 = jnp.where(live, M1, M0[b])
        M_scr[b] = M1
        Mf_ref[b] = M1

    V_all = jnp.concatenate(Vb, axis=0)

    def retrieve(Qa, n):
        rows = jax.lax.broadcasted_iota(jnp.int32, (n * C, C), 0)
        cols = jax.lax.broadcasted_iota(jnp.int32, (n * C, C), 1)
        causal = cols <= (rows & (C - 1))
        outs = []
        for b in range(B):
            Qg = Qa[b * n * C:(b + 1) * n * C]
            P = jnp.where(causal, _dotT(Qg, Kb[b], 1, 1), 0.0)
            outs.append(_f32dot(P, Ub[b]) + _dotT(Qg, M0[b], 1, 1))
        return jnp.concatenate(outs, axis=0)

    cur, n = V_all, 1
    levels = [V_all]
    for _ in range(depth):
        Ps = [_f32dot(cur, Q_ref[r]) for r in range(R)]
        pieces = [Ps[r][(b * n + p) * C:(b * n + p + 1) * C]
                  for b in range(B) for p in range(n) for r in range(R)]
        Qa = _l2n(jnp.concatenate(pieces, axis=0))
        n *= R
        cur = retrieve(Qa, n)
        levels.append(cur)
    all_slots = jnp.concatenate(levels, axis=0)
    out_all = _f32dot(all_slots, WoutT_ref[...])

    base, slot = 0, 0
    for lvl in range(depth + 1):
        n = R ** lvl
        for b in range(B):
            for p in range(n):
                seg = base + (b * n + p) * C
                out_ref[b, :, slot + p, :] = out_all[seg:seg + C]
        base += B * n * C
        slot += n


def kernel(x, M, Wv, Q, Wk, Wout):
    B, S, E = x.shape
    D = M.shape[1]
    R = Q.shape[0]
    C = _C
    nc = S // C
    nslots = 1
    k = 1
    for _ in range(_DEPTH):
        k *= R
        nslots += k
    out, Mf = pl.pallas_call(
        functools.partial(_amem_kernel, B=B, C=C, R=R, depth=_DEPTH),
        grid=(nc + 1,),
        in_specs=[
            pl.BlockSpec((B, C, E), lambda c: (0, jnp.minimum(c, nc - 1), 0)),
            pl.BlockSpec((B, D, D), lambda c: (0, 0, 0)),
            pl.BlockSpec((E, D), lambda c: (0, 0)),
            pl.BlockSpec((E, D), lambda c: (0, 0)),
            pl.BlockSpec((R, D, D), lambda c: (0, 0, 0)),
            pl.BlockSpec((D, E), lambda c: (0, 0)),
        ],
        out_specs=[
            pl.BlockSpec((B, C, nslots, E),
                         lambda c: (0, jnp.maximum(c - 1, 0), 0, 0)),
            pl.BlockSpec((B, D, D), lambda c: (0, 0, 0)),
        ],
        out_shape=[
            jax.ShapeDtypeStruct((B, S, nslots, E), jnp.float32),
            jax.ShapeDtypeStruct((B, D, D), jnp.float32),
        ],
        scratch_shapes=[
            pltpu.VMEM((B, D, D), jnp.float32),
            pltpu.VMEM((2, B, C, D), jnp.float32),
            pltpu.VMEM((2, B, C, D), jnp.float32),
            pltpu.VMEM((2, B, C, C), jnp.float32),
        ],
        compiler_params=pltpu.CompilerParams(
            dimension_semantics=("arbitrary",),
        ),
    )(x, M, Wv.T, Wk.T, Q, Wout.T)
    return out, Mf
